# Initial kernel scaffold; baseline (speedup 1.0000x reference)
import jax
import jax.numpy as jnp
from jax.experimental import pallas as pl


def kernel(current_repr, session_ids, item_emb, feature_queue, session_queue, target_queue, Wq, bq, Wk, bk, W1, b1, W2, b2, fallback_context, fallback_summary):
    raise NotImplementedError("write your pallas kernel here")


# speedup vs baseline: 6.6307x; 6.6307x over previous
import functools

import jax
import jax.numpy as jnp
import numpy as np
from jax import lax
from jax.experimental import pallas as pl
from jax.experimental.pallas import tpu as pltpu

_B = 1024
_D = 128
_CD = 256
_M = 100000
_TOPK = 16
_TEMP = 0.07
_S = 16
_T = 2048
_NT = (_M + _T - 1) // _T
_MP = _NT * _T
_C = _MP // _S
_NEG = float("-inf")


def _qproj_body(cr_ref, wq_ref, bq_ref, q_ref):
    q = jnp.dot(cr_ref[...], wq_ref[...].T, preferred_element_type=jnp.float32)
    q = q + bq_ref[...]
    n = jnp.sqrt(jnp.sum(q * q, axis=1, keepdims=True))
    q_ref[...] = q / jnp.maximum(n, 1e-12)


def _qproj(cr, wq, bq):
    return pl.pallas_call(
        _qproj_body,
        out_shape=jax.ShapeDtypeStruct((_B, _D), jnp.float32),
    )(cr, wq, bq.reshape(1, _D))


_BB2 = 512


def _sims_body(q_ref, fq_ref, wk_ref, bk_ref, sid_ref, sq_ref,
               sims_ref, cmax_ref):
    i = pl.program_id(1)
    k = jnp.dot(fq_ref[...], wk_ref[...].T, preferred_element_type=jnp.float32)
    k = k + bk_ref[...]
    n = jnp.sqrt(jnp.sum(k * k, axis=1, keepdims=True))
    k = k / jnp.maximum(n, 1e-12)
    sims = jnp.dot(q_ref[...], k.T, preferred_element_type=jnp.float32)
    col = i * _T + lax.broadcasted_iota(jnp.int32, (1, _T), 1)
    valid = (sq_ref[0] != sid_ref[...]) & (col < _M)
    sims = jnp.where(valid, sims, _NEG)
    sims_ref[...] = sims
    cmax_ref[...] = jnp.max(sims.reshape(_BB2, _S, _T // _S), axis=1)


def _sims_cmax(q, fq_pad, wk, bk, sid, sq_pad):
    return pl.pallas_call(
        _sims_body,
        grid=(_B // _BB2, _NT),
        in_specs=[
            pl.BlockSpec((_BB2, _D), lambda b, i: (b, 0)),
            pl.BlockSpec((_T, _D), lambda b, i: (i, 0)),
            pl.BlockSpec((_D, _D), lambda b, i: (0, 0)),
            pl.BlockSpec((1, _D), lambda b, i: (0, 0)),
            pl.BlockSpec((_BB2, 1), lambda b, i: (b, 0)),
            pl.BlockSpec((1, 1, _T), lambda b, i: (i, 0, 0)),
        ],
        out_specs=[
            pl.BlockSpec((_BB2, _T), lambda b, i: (b, i)),
            pl.BlockSpec((_BB2, _T // _S), lambda b, i: (b, i)),
        ],
        out_shape=[
            jax.ShapeDtypeStruct((_B, _MP), jnp.float32),
            jax.ShapeDtypeStruct((_B, _C), jnp.float32),
        ],
    )(q, fq_pad, wk, bk.reshape(1, _D), sid, sq_pad)


_BB3 = 128


def _topchunk_body(cmax_ref, cid_ref):
    x = cmax_ref[...]
    iota = lax.broadcasted_iota(jnp.int32, (_BB3, _C), 1)
    avail = jnp.ones((_BB3, _C), jnp.bool_)
    for j in range(_TOPK):
        xa = jnp.where(avail, x, _NEG)
        m = jnp.max(xa, axis=1, keepdims=True)
        eq = avail & ((xa == m) | (m == _NEG))
        idx = jnp.min(jnp.where(eq, iota, _C), axis=1, keepdims=True)
        cid_ref[:, j] = idx[:, 0]
        avail = avail & (iota != idx)


def _topchunks(cmax):
    return pl.pallas_call(
        _topchunk_body,
        grid=(_B // _BB3,),
        in_specs=[pl.BlockSpec((_BB3, _C), lambda i: (i, 0))],
        out_specs=pl.BlockSpec((_BB3, _TOPK), lambda i: (i, 0)),
        out_shape=jax.ShapeDtypeStruct((_B, _TOPK), jnp.int32),
    )(cmax)


def _candidates_jnp(sims, cids, fq, tq, ie):
    gidx = ((cids // 128) * _T + (cids % 128))[:, :, None] \
        + 128 * jnp.arange(_S, dtype=jnp.int32)
    gidx = gidx.reshape(_B, _TOPK * _S)
    cand = jnp.take_along_axis(sims, gidx, axis=1)
    tv, sel = lax.top_k(cand, _TOPK)
    ti = jnp.take_along_axis(gidx, sel, axis=1)
    ti = jnp.minimum(ti, _M - 1)
    nf = jnp.take(fq, ti, axis=0)
    tgt = jnp.take(tq, ti, axis=0)
    iemb = jnp.take(ie, jnp.maximum(tgt, 0), axis=0)
    return tv, nf, iemb


_BB5 = 128


def _final_body(tv_ref, nf_ref, ie_ref, w1_ref, b1_ref, w2_ref, b2_ref,
                fc_ref, fs_ref, ctx_ref, sum_ref, used_ref):
    tv = tv_ref[...]
    selected = tv > -1e30
    used = jnp.any(selected, axis=1, keepdims=True)
    logits = jnp.where(selected, tv * (1.0 / _TEMP), -1e9)
    m = jnp.max(logits, axis=1, keepdims=True)
    e = jnp.exp(logits - m)
    a = e / jnp.sum(e, axis=1, keepdims=True)
    a = a * selected.astype(jnp.float32)
    a = a / jnp.maximum(jnp.sum(a, axis=1, keepdims=True), 1e-12)

    nf = nf_ref[...]
    ctx = jnp.sum(a[:, :, None] * nf, axis=1)

    su = jnp.concatenate([nf, ie_ref[...]], axis=2).reshape(_BB5 * _TOPK, 2 * _D)
    h = jnp.dot(su, w1_ref[...].T, preferred_element_type=jnp.float32) + b1_ref[...]
    h = 0.5 * h * (1.0 + lax.erf(h * np.float32(1.0 / np.sqrt(2.0))))
    sv = jnp.dot(h, w2_ref[...].T, preferred_element_type=jnp.float32) + b2_ref[...]
    sv = sv.reshape(_BB5, _TOPK, _CD)
    summ = jnp.sum(a[:, :, None] * sv, axis=1)

    ctx_ref[...] = jnp.where(used, ctx, fc_ref[...])
    sum_ref[...] = jnp.where(used, summ, fs_ref[...])
    used_ref[...] = used.astype(jnp.int32)


def _final(tv, nf, iemb, w1, b1, w2, b2, fc, fs):
    return pl.pallas_call(
        _final_body,
        grid=(_B // _BB5,),
        in_specs=[
            pl.BlockSpec((_BB5, _TOPK), lambda i: (i, 0)),
            pl.BlockSpec((_BB5, _TOPK, _D), lambda i: (i, 0, 0)),
            pl.BlockSpec((_BB5, _TOPK, _D), lambda i: (i, 0, 0)),
            pl.BlockSpec((_CD, 2 * _D), lambda i: (0, 0)),
            pl.BlockSpec((1, _CD), lambda i: (0, 0)),
            pl.BlockSpec((_CD, _CD), lambda i: (0, 0)),
            pl.BlockSpec((1, _CD), lambda i: (0, 0)),
            pl.BlockSpec((1, _D), lambda i: (0, 0)),
            pl.BlockSpec((1, _CD), lambda i: (0, 0)),
        ],
        out_specs=[
            pl.BlockSpec((_BB5, _D), lambda i: (i, 0)),
            pl.BlockSpec((_BB5, _CD), lambda i: (i, 0)),
            pl.BlockSpec((_BB5, 1), lambda i: (i, 0)),
        ],
        out_shape=[
            jax.ShapeDtypeStruct((_B, _D), jnp.float32),
            jax.ShapeDtypeStruct((_B, _CD), jnp.float32),
            jax.ShapeDtypeStruct((_B, 1), jnp.int32),
        ],
    )(tv, nf, iemb, w1, b1.reshape(1, _CD), w2, b2.reshape(1, _CD),
      fc.reshape(1, _D), fs.reshape(1, _CD))


def kernel(current_repr, session_ids, item_emb, feature_queue, session_queue,
           target_queue, Wq, bq, Wk, bk, W1, b1, W2, b2,
           fallback_context, fallback_summary):
    q = _qproj(current_repr, Wq, bq)
    fq_pad = jnp.pad(feature_queue, ((0, _MP - _M), (0, 0)))
    sq_pad = jnp.pad(session_queue, (0, _MP - _M)).reshape(_NT, 1, _T)
    sid = session_ids.astype(jnp.int32).reshape(_B, 1)
    sims, cmax = _sims_cmax(q, fq_pad, Wk, bk, sid, sq_pad)
    cids = _topchunks(cmax)
    tv, nf, iemb = _candidates_jnp(sims, cids, feature_queue,
                                   target_queue, item_emb)
    ctx, summ, used = _final(tv, nf, iemb, W1, b1, W2, b2,
                             fallback_context, fallback_summary)
    return ctx, summ, used[:, 0] != 0
